# baseline (speedup 1.0000x reference)
import functools

import jax
import jax.numpy as jnp
from jax.experimental import pallas as pl
from jax.experimental.pallas import tpu as pltpu

N = 10000
E = 320000
F = 128
ED = 16
AVG_D_LOG = 3.4965


def _ab_kernel(h_ref, ws_ref, wd_ref, b_ref, a_ref, b_out_ref):
    h = h_ref[...]
    a_ref[...] = jnp.dot(h, ws_ref[...], preferred_element_type=jnp.float32)
    b_out_ref[...] = (
        jnp.dot(h, wd_ref[...], preferred_element_type=jnp.float32)
        + b_ref[...]
    )


def _c_kernel(ea_ref, we_ref, c_ref):
    c_ref[...] = jnp.dot(ea_ref[...], we_ref[...],
                         preferred_element_type=jnp.float32)


def kernel(h, edge_index, edge_attr, W_pre, b_pre, W_post, b_post, gamma, beta):
    src = edge_index[0]
    dst = edge_index[1]
    Ws = W_pre[:F]
    Wd = W_pre[F:2 * F]
    We = W_pre[2 * F:]

    a_tab, b_tab = pl.pallas_call(
        _ab_kernel,
        out_shape=(
            jax.ShapeDtypeStruct((N, F), jnp.float32),
            jax.ShapeDtypeStruct((N, F), jnp.float32),
        ),
    )(h, Ws, Wd, b_pre.reshape(1, F))

    EB = 3200
    c_tab = pl.pallas_call(
        _c_kernel,
        grid=(E // EB,),
        in_specs=[
            pl.BlockSpec((EB, ED), lambda i: (i, 0)),
            pl.BlockSpec((ED, F), lambda i: (0, 0)),
        ],
        out_specs=pl.BlockSpec((EB, F), lambda i: (i, 0)),
        out_shape=jax.ShapeDtypeStruct((E, F), jnp.float32),
    )(edge_attr, We)

    e = jax.nn.relu(a_tab[src] + b_tab[dst] + c_tab)

    deg = jax.ops.segment_sum(jnp.ones((E,), jnp.float32), dst, num_segments=N)
    deg_safe = jnp.maximum(deg, 1.0)
    mean = jax.ops.segment_sum(e, dst, num_segments=N) / deg_safe[:, None]
    mx = jax.ops.segment_max(e, dst, num_segments=N)
    mx = jnp.where(deg[:, None] > 0, mx, 0.0)
    mn = -jax.ops.segment_max(-e, dst, num_segments=N)
    mn = jnp.where(deg[:, None] > 0, mn, 0.0)
    sq_mean = jax.ops.segment_sum(e * e, dst, num_segments=N) / deg_safe[:, None]
    std = jnp.sqrt(jax.nn.relu(sq_mean - mean * mean) + 1e-5)

    agg = jnp.concatenate([mean, mx, mn, std], axis=1)
    logd = jnp.log(deg_safe + 1.0)
    amp = agg * (logd / AVG_D_LOG)[:, None]
    att = agg * (AVG_D_LOG / logd)[:, None]
    hcat = jnp.concatenate([h, agg, amp, att], axis=1)

    out = jax.nn.relu(hcat @ W_post + b_post)
    mu = jnp.mean(out, axis=0)
    var = jnp.var(out, axis=0)
    return (out - mu) / jnp.sqrt(var + 1e-5) * gamma + beta
